# Initial kernel scaffold; baseline (speedup 1.0000x reference)
import jax
import jax.numpy as jnp
from jax.experimental import pallas as pl


def kernel(seq_in_embeds, W):
    raise NotImplementedError("write your pallas kernel here")


# speedup vs baseline: 2.2006x; 2.2006x over previous
import functools

import jax
import jax.numpy as jnp
from jax import lax
from jax.experimental import pallas as pl
from jax.experimental.pallas import tpu as pltpu
from jax.experimental.pallas import tpu_sc as plsc

_BATCH = 4
_ROWS = 2048
_FEAT = 1024
_NUM_CORES = 2
_NUM_SUBCORES = 16
_NUM_WORKERS = _NUM_CORES * _NUM_SUBCORES
_ROWS_PER_WORKER = _ROWS // _NUM_WORKERS


@jax.jit
def _broadcast_table(w):
    mesh = plsc.VectorSubcoreMesh(core_axis_name="c", subcore_axis_name="s")

    @functools.partial(
        pl.kernel,
        mesh=mesh,
        out_type=jax.ShapeDtypeStruct((_BATCH, _ROWS, _FEAT), jnp.float32),
        scratch_types=[
            pltpu.VMEM((_ROWS_PER_WORKER, _FEAT), jnp.float32),
            pltpu.SemaphoreType.DMA,
        ],
    )
    def k(w_hbm, out_hbm, buf, sem):
        wid = lax.axis_index("s") * _NUM_CORES + lax.axis_index("c")
        base = wid * _ROWS_PER_WORKER
        pltpu.sync_copy(w_hbm.at[pl.ds(base, _ROWS_PER_WORKER)], buf)
        copies = [
            pltpu.async_copy(
                buf, out_hbm.at[b, pl.ds(base, _ROWS_PER_WORKER)], sem
            )
            for b in range(_BATCH)
        ]
        for c in copies:
            c.wait()

    return k(w)


def kernel(seq_in_embeds, W):
    del seq_in_embeds
    return _broadcast_table(W)
